# baseline (speedup 1.0000x reference)
import functools

import jax
import jax.numpy as jnp
from jax import lax
from jax.experimental import pallas as pl
from jax.experimental.pallas import tpu as pltpu
from jax.experimental.pallas import tpu_sc as plsc

INDEX = 512
HEAD_NUM = 12
B = 2
S = 2048
D = 768
K_TOP = INDEX - 2
N_SEL = K_TOP + 1
N_OTHER = S - INDEX + 1

ROWS_BLK = 2048
NSUB = 16
NCORE = 2
NW = NSUB * NCORE
RPW = INDEX // NW


def _colsum_kernel(a_ref, o_ref):
    bh = pl.program_id(0)
    r = pl.program_id(1)
    blk = a_ref[0]
    row0 = r * ROWS_BLK
    i_idx = lax.broadcasted_iota(jnp.int32, (ROWS_BLK, S), 0) + row0
    j_idx = lax.broadcasted_iota(jnp.int32, (ROWS_BLK, S), 1)
    masked = jnp.where(i_idx == j_idx, 0.0, blk)
    contrib = jnp.sum(masked, axis=0, keepdims=True)

    @pl.when(jnp.logical_and(lax.rem(bh, HEAD_NUM) == 0, r == 0))
    def _():
        o_ref[...] = jnp.zeros_like(o_ref)

    o_ref[0] += contrib


def _attended(atten):
    return pl.pallas_call(
        _colsum_kernel,
        grid=(B * HEAD_NUM, S // ROWS_BLK),
        in_specs=[pl.BlockSpec((1, ROWS_BLK, S), lambda bh, r: (bh, r, 0))],
        out_specs=pl.BlockSpec((1, 1, S), lambda bh, r: (bh // HEAD_NUM, 0, 0)),
        out_shape=jax.ShapeDtypeStruct((B, 1, S), jnp.float32),
        compiler_params=pltpu.CompilerParams(
            dimension_semantics=("arbitrary", "arbitrary"),
        ),
    )(atten)


def _select_kernel(a_ref, idx_ref, e_ref):
    a = a_ref[0] * (1.0 / HEAD_NUM)
    jvec = lax.broadcasted_iota(jnp.int32, (1, S), 1)
    valid = jvec >= 1

    u = lax.bitcast_convert_type(a, jnp.uint32)
    key = jnp.where((u >> 31) == 1, ~u, u | jnp.uint32(0x80000000))
    key = jnp.where(valid, key, jnp.uint32(0))

    def body(k, t):
        cand = t | (jnp.uint32(1) << (jnp.uint32(31) - k.astype(jnp.uint32)))
        cnt = jnp.sum((key >= cand).astype(jnp.int32))
        return jnp.where(cnt >= K_TOP, cand, t)

    thr = lax.fori_loop(0, 32, body, jnp.uint32(0))

    def prefix_excl(v):
        acc = v
        for k in (1, 2, 4, 8, 16, 32, 64, 128, 256, 512, 1024):
            acc = acc + jnp.where(jvec >= k, pltpu.roll(acc, k, 1), 0.0)
        return acc - v

    gt = jnp.logical_and(key > thr, valid)
    eq = jnp.logical_and(key == thr, valid)
    n_gt = jnp.sum(gt.astype(jnp.int32))
    need_eq = K_TOP - n_gt
    eq_f = eq.astype(jnp.float32)
    eq_pref = prefix_excl(eq_f)
    sel_rest = jnp.logical_or(
        gt, jnp.logical_and(eq, eq_pref < need_eq.astype(jnp.float32))
    )
    sel_full = jnp.logical_or(sel_rest, jvec == 0)

    sel_f = sel_full.astype(jnp.float32)
    pos = prefix_excl(sel_f)

    other = jnp.logical_and(valid, jnp.logical_not(sel_rest))
    m = jnp.max(jnp.where(other, a, -jnp.inf))
    e_ref[0] = jnp.where(other, jnp.exp(a - m), 0.0)

    prow = lax.broadcasted_iota(jnp.int32, (INDEX, S), 0)
    onehot = jnp.logical_and(prow == pos.astype(jnp.int32), sel_full)
    idx_f = lax.dot_general(
        jvec.astype(jnp.float32), onehot.astype(jnp.float32),
        (((1,), (1,)), ((), ())),
        precision=lax.Precision.HIGHEST,
        preferred_element_type=jnp.float32,
    )
    idx_ref[0] = idx_f.astype(jnp.int32)


def _select(attended):
    return pl.pallas_call(
        _select_kernel,
        grid=(B,),
        in_specs=[pl.BlockSpec((1, 1, S), lambda b: (b, 0, 0))],
        out_specs=(
            pl.BlockSpec((1, 1, INDEX), lambda b: (b, 0, 0)),
            pl.BlockSpec((1, 1, S), lambda b: (b, 0, 0)),
        ),
        out_shape=(
            jax.ShapeDtypeStruct((B, 1, INDEX), jnp.int32),
            jax.ShapeDtypeStruct((B, 1, S), jnp.float32),
        ),
        compiler_params=pltpu.CompilerParams(
            dimension_semantics=("arbitrary",),
        ),
    )(attended)


def _sc_gather(idx, x):
    mesh = plsc.VectorSubcoreMesh(core_axis_name="c", subcore_axis_name="s")

    @functools.partial(
        pl.kernel,
        out_type=jax.ShapeDtypeStruct((B, INDEX, D), jnp.float32),
        mesh=mesh,
        scratch_types=[
            pltpu.VMEM((B * RPW,), jnp.int32),
            pltpu.VMEM((B * RPW, D), jnp.float32),
            pltpu.SemaphoreType.DMA,
        ],
    )
    def sc_kernel(idx_hbm, x_hbm, ext_hbm, my_idx, rows_v, sem):
        c = lax.axis_index("c")
        s = lax.axis_index("s")
        wid = s * NCORE + c
        base = wid * RPW
        for b in range(B):
            pltpu.sync_copy(
                idx_hbm.at[b, 0, pl.ds(base, RPW)],
                my_idx.at[pl.ds(b * RPW, RPW)],
            )
        cp0 = pltpu.async_copy(x_hbm.at[0].at[my_idx.at[pl.ds(0, RPW)]],
                               rows_v.at[pl.ds(0, RPW)], sem)
        cp1 = pltpu.async_copy(x_hbm.at[1].at[my_idx.at[pl.ds(RPW, RPW)]],
                               rows_v.at[pl.ds(RPW, RPW)], sem)
        cp0.wait()
        cp1.wait()
        for b in range(B):
            pltpu.sync_copy(
                rows_v.at[pl.ds(b * RPW, RPW)],
                ext_hbm.at[b, pl.ds(base, RPW)],
            )

    return sc_kernel(idx, x)


def _wsum_kernel(e_ref, x_ref, o_ref):
    e = e_ref[0]
    z = jnp.sum(e)
    w = e / (z * N_OTHER)
    o_ref[0] = lax.dot_general(
        w, x_ref[0],
        (((1,), (0,)), ((), ())),
        precision=lax.Precision.HIGHEST,
        preferred_element_type=jnp.float32,
    )


def _wsum(e, x):
    return pl.pallas_call(
        _wsum_kernel,
        grid=(B,),
        in_specs=[
            pl.BlockSpec((1, 1, S), lambda b: (b, 0, 0)),
            pl.BlockSpec((1, S, D), lambda b: (b, 0, 0)),
        ],
        out_specs=pl.BlockSpec((1, 1, D), lambda b: (b, 0, 0)),
        out_shape=jax.ShapeDtypeStruct((B, 1, D), jnp.float32),
        compiler_params=pltpu.CompilerParams(
            dimension_semantics=("arbitrary",),
        ),
    )(e, x)


@jax.jit
def kernel(x, atten):
    attended = _attended(atten)
    idx, e = _select(attended)
    ext = _sc_gather(idx, x)
    wsum = _wsum(e, x)
    return jnp.concatenate([ext[:, :N_SEL], wsum], axis=1)


# speedup vs baseline: 1.0624x; 1.0624x over previous
import functools

import jax
import jax.numpy as jnp
from jax import lax
from jax.experimental import pallas as pl
from jax.experimental.pallas import tpu as pltpu
from jax.experimental.pallas import tpu_sc as plsc

INDEX = 512
HEAD_NUM = 12
B = 2
S = 2048
D = 768
K_TOP = INDEX - 2
N_SEL = K_TOP + 1
N_OTHER = S - INDEX + 1

ROWS_BLK = 2048
NSUB = 16
RPW = INDEX // NSUB


def _fused_kernel(a_ref, x_ref, idx_ref, ws_ref, att_ref):
    bh = pl.program_id(0)
    r = pl.program_id(1)
    h = lax.rem(bh, HEAD_NUM)
    blk = a_ref[0]
    row0 = r * ROWS_BLK
    i_idx = lax.broadcasted_iota(jnp.int32, (ROWS_BLK, S), 0) + row0
    j_idx = lax.broadcasted_iota(jnp.int32, (ROWS_BLK, S), 1)
    masked = jnp.where(i_idx == j_idx, 0.0, blk)
    contrib = jnp.sum(masked, axis=0, keepdims=True)

    @pl.when(jnp.logical_and(h == 0, r == 0))
    def _():
        att_ref[...] = jnp.zeros_like(att_ref)

    att_ref[...] += contrib

    @pl.when(jnp.logical_and(h == HEAD_NUM - 1, r == S // ROWS_BLK - 1))
    def _select():
        a = att_ref[...] * (1.0 / HEAD_NUM)
        jvec = lax.broadcasted_iota(jnp.int32, (1, S), 1)
        valid = jvec >= 1

        u = lax.bitcast_convert_type(a, jnp.uint32)
        key = jnp.where((u >> 31) == 1, ~u, u | jnp.uint32(0x80000000))
        key = jnp.where(valid, key, jnp.uint32(0))

        def body(k, t):
            cand = t | (jnp.uint32(1) << (jnp.uint32(31) - k.astype(jnp.uint32)))
            cnt = jnp.sum((key >= cand).astype(jnp.int32))
            return jnp.where(cnt >= K_TOP, cand, t)

        thr = lax.fori_loop(0, 32, body, jnp.uint32(0))

        def prefix_excl(v):
            acc = v
            for k in (1, 2, 4, 8, 16, 32, 64, 128, 256, 512, 1024):
                acc = acc + jnp.where(jvec >= k, pltpu.roll(acc, k, 1), 0.0)
            return acc - v

        gt = jnp.logical_and(key > thr, valid)
        eq = jnp.logical_and(key == thr, valid)
        n_gt = jnp.sum(gt.astype(jnp.int32))
        need_eq = K_TOP - n_gt
        eq_f = eq.astype(jnp.float32)
        eq_pref = prefix_excl(eq_f)
        sel_rest = jnp.logical_or(
            gt, jnp.logical_and(eq, eq_pref < need_eq.astype(jnp.float32))
        )
        sel_full = jnp.logical_or(sel_rest, jvec == 0)

        sel_f = sel_full.astype(jnp.float32)
        pos = prefix_excl(sel_f)

        other = jnp.logical_and(valid, jnp.logical_not(sel_rest))
        m = jnp.max(jnp.where(other, a, -jnp.inf))
        e = jnp.where(other, jnp.exp(a - m), 0.0)
        w = e / (jnp.sum(e) * N_OTHER)
        ws_ref[0] = lax.dot_general(
            w, x_ref[0],
            (((1,), (0,)), ((), ())),
            precision=lax.Precision.HIGHEST,
            preferred_element_type=jnp.float32,
        )

        prow = lax.broadcasted_iota(jnp.int32, (INDEX, S), 0)
        onehot = jnp.logical_and(prow == pos.astype(jnp.int32), sel_full)
        idx_f = lax.dot_general(
            jvec.astype(jnp.float32), onehot.astype(jnp.float32),
            (((1,), (1,)), ((), ())),
            precision=lax.Precision.HIGHEST,
            preferred_element_type=jnp.float32,
        )
        idx_ref[0] = idx_f.astype(jnp.int32)


def _select(atten, x):
    return pl.pallas_call(
        _fused_kernel,
        grid=(B * HEAD_NUM, S // ROWS_BLK),
        in_specs=[
            pl.BlockSpec((1, ROWS_BLK, S), lambda bh, r: (bh, r, 0)),
            pl.BlockSpec((1, S, D), lambda bh, r: (bh // HEAD_NUM, 0, 0)),
        ],
        out_specs=(
            pl.BlockSpec((1, 1, INDEX), lambda bh, r: (bh // HEAD_NUM, 0, 0)),
            pl.BlockSpec((1, 1, D), lambda bh, r: (bh // HEAD_NUM, 0, 0)),
        ),
        out_shape=(
            jax.ShapeDtypeStruct((B, 1, INDEX), jnp.int32),
            jax.ShapeDtypeStruct((B, 1, D), jnp.float32),
        ),
        scratch_shapes=[pltpu.VMEM((1, S), jnp.float32)],
        compiler_params=pltpu.CompilerParams(
            dimension_semantics=("arbitrary", "arbitrary"),
        ),
    )(atten, x)


def _sc_gather(idx, ws, x):
    mesh = plsc.VectorSubcoreMesh(core_axis_name="c", subcore_axis_name="s")

    @functools.partial(
        pl.kernel,
        out_type=jax.ShapeDtypeStruct((B, INDEX, D), jnp.float32),
        mesh=mesh,
        scratch_types=[
            pltpu.VMEM((RPW,), jnp.int32),
            pltpu.VMEM((RPW, D), jnp.float32),
            pltpu.SemaphoreType.DMA,
        ],
    )
    def sc_kernel(idx_hbm, ws_hbm, x_hbm, out_hbm, my_idx, rows_v, sem):
        c = lax.axis_index("c")
        s = lax.axis_index("s")
        base = s * RPW
        pltpu.sync_copy(idx_hbm.at[c, 0, pl.ds(base, RPW)], my_idx)
        pltpu.async_copy(x_hbm.at[c].at[my_idx], rows_v, sem).wait()

        @pl.when(s == NSUB - 1)
        def _():
            pltpu.sync_copy(ws_hbm.at[c, 0], rows_v.at[RPW - 1])

        pltpu.sync_copy(rows_v, out_hbm.at[c, pl.ds(base, RPW)])

    return sc_kernel(idx, ws, x)


@jax.jit
def kernel(x, atten):
    idx, ws = _select(atten, x)
    return _sc_gather(idx, ws, x)


# speedup vs baseline: 1.1059x; 1.0410x over previous
import functools

import jax
import jax.numpy as jnp
from jax import lax
from jax.experimental import pallas as pl
from jax.experimental.pallas import tpu as pltpu
from jax.experimental.pallas import tpu_sc as plsc

INDEX = 512
HEAD_NUM = 12
B = 2
S = 2048
D = 768
K_TOP = INDEX - 2
N_SEL = K_TOP + 1
N_OTHER = S - INDEX + 1

ROWS_BLK = 2048
NSUB = 16
RPW = INDEX // NSUB


def _fused_kernel(a_ref, x_ref, idx_ref, ws_ref, att_ref):
    bh = pl.program_id(0)
    r = pl.program_id(1)
    h = lax.rem(bh, HEAD_NUM)
    blk = a_ref[0]
    row0 = r * ROWS_BLK
    i_idx = lax.broadcasted_iota(jnp.int32, (ROWS_BLK, S), 0) + row0
    j_idx = lax.broadcasted_iota(jnp.int32, (ROWS_BLK, S), 1)
    masked = jnp.where(i_idx == j_idx, 0.0, blk)
    contrib = jnp.sum(masked, axis=0, keepdims=True)

    @pl.when(jnp.logical_and(h == 0, r == 0))
    def _():
        att_ref[...] = jnp.zeros_like(att_ref)

    att_ref[...] += contrib

    @pl.when(jnp.logical_and(h == HEAD_NUM - 1, r == S // ROWS_BLK - 1))
    def _select():
        a = att_ref[...] * (1.0 / HEAD_NUM)
        jvec = lax.broadcasted_iota(jnp.int32, (1, S), 1)
        valid = jvec >= 1

        u = lax.bitcast_convert_type(a, jnp.uint32)
        key = jnp.where((u >> 31) == 1, ~u, u | jnp.uint32(0x80000000))
        key = jnp.where(valid, key, jnp.uint32(0))

        def body(k, t):
            cand = t | (jnp.uint32(1) << (jnp.uint32(31) - k.astype(jnp.uint32)))
            cnt = jnp.sum((key >= cand).astype(jnp.int32))
            return jnp.where(cnt >= K_TOP, cand, t)

        thr = lax.fori_loop(0, 32, body, jnp.uint32(0))

        def prefix_excl(v):
            acc = v
            for k in (1, 2, 4, 8, 16, 32, 64, 128, 256, 512, 1024):
                acc = acc + jnp.where(jvec >= k, pltpu.roll(acc, k, 1), 0.0)
            return acc - v

        gt = jnp.logical_and(key > thr, valid)
        eq = jnp.logical_and(key == thr, valid)
        n_gt = jnp.sum(gt.astype(jnp.int32))
        need_eq = K_TOP - n_gt
        eq_f = eq.astype(jnp.float32)
        eq_pref = prefix_excl(eq_f)
        sel_rest = jnp.logical_or(
            gt, jnp.logical_and(eq, eq_pref < need_eq.astype(jnp.float32))
        )
        sel_full = jnp.logical_or(sel_rest, jvec == 0)

        sel_f = sel_full.astype(jnp.float32)
        pos = prefix_excl(sel_f)

        other = jnp.logical_and(valid, jnp.logical_not(sel_rest))
        m = jnp.max(jnp.where(other, a, -jnp.inf))
        e = jnp.where(other, jnp.exp(a - m), 0.0)
        w = e / (jnp.sum(e) * N_OTHER)
        ws_ref[0] = lax.dot_general(
            w, x_ref[0],
            (((1,), (0,)), ((), ())),
            precision=lax.Precision.HIGHEST,
            preferred_element_type=jnp.float32,
        )

        prow = lax.broadcasted_iota(jnp.int32, (INDEX, S), 0)
        onehot = jnp.logical_and(prow == pos.astype(jnp.int32), sel_full)
        jhi = (jvec >> 7).astype(jnp.float32)
        jlo = (jvec & 127).astype(jnp.float32)
        jmat = jnp.concatenate([jhi, jlo], axis=0)
        hl = lax.dot_general(
            jmat, onehot.astype(jnp.float32),
            (((1,), (1,)), ((), ())),
            preferred_element_type=jnp.float32,
        )
        idx_ref[0] = (hl[0:1] * 128.0 + hl[1:2]).astype(jnp.int32)


def _select(atten, x):
    return pl.pallas_call(
        _fused_kernel,
        grid=(B * HEAD_NUM, S // ROWS_BLK),
        in_specs=[
            pl.BlockSpec((1, ROWS_BLK, S), lambda bh, r: (bh, r, 0)),
            pl.BlockSpec((1, S, D), lambda bh, r: (bh // HEAD_NUM, 0, 0)),
        ],
        out_specs=(
            pl.BlockSpec((1, 1, INDEX), lambda bh, r: (bh // HEAD_NUM, 0, 0)),
            pl.BlockSpec((1, 1, D), lambda bh, r: (bh // HEAD_NUM, 0, 0)),
        ),
        out_shape=(
            jax.ShapeDtypeStruct((B, 1, INDEX), jnp.int32),
            jax.ShapeDtypeStruct((B, 1, D), jnp.float32),
        ),
        scratch_shapes=[pltpu.VMEM((1, S), jnp.float32)],
        compiler_params=pltpu.CompilerParams(
            dimension_semantics=("arbitrary", "arbitrary"),
        ),
    )(atten, x)


def _sc_gather(idx, ws, x):
    mesh = plsc.VectorSubcoreMesh(core_axis_name="c", subcore_axis_name="s")

    @functools.partial(
        pl.kernel,
        out_type=jax.ShapeDtypeStruct((B, INDEX, D), jnp.float32),
        mesh=mesh,
        scratch_types=[
            pltpu.VMEM((RPW,), jnp.int32),
            pltpu.VMEM((RPW, D), jnp.float32),
            pltpu.SemaphoreType.DMA,
        ],
    )
    def sc_kernel(idx_hbm, ws_hbm, x_hbm, out_hbm, my_idx, rows_v, sem):
        c = lax.axis_index("c")
        s = lax.axis_index("s")
        base = s * RPW
        pltpu.sync_copy(idx_hbm.at[c, 0, pl.ds(base, RPW)], my_idx)
        pltpu.async_copy(x_hbm.at[c].at[my_idx], rows_v, sem).wait()

        @pl.when(s == NSUB - 1)
        def _():
            pltpu.sync_copy(ws_hbm.at[c, 0], rows_v.at[RPW - 1])

        pltpu.sync_copy(rows_v, out_hbm.at[c, pl.ds(base, RPW)])

    return sc_kernel(idx, ws, x)


@jax.jit
def kernel(x, atten):
    idx, ws = _select(atten, x)
    return _sc_gather(idx, ws, x)


# speedup vs baseline: 1.1240x; 1.0163x over previous
import functools

import jax
import jax.numpy as jnp
from jax import lax
from jax.experimental import pallas as pl
from jax.experimental.pallas import tpu as pltpu
from jax.experimental.pallas import tpu_sc as plsc

INDEX = 512
HEAD_NUM = 12
B = 2
S = 2048
D = 768
K_TOP = INDEX - 2
N_SEL = K_TOP + 1
N_OTHER = S - INDEX + 1

ROWS_BLK = 2048
NSUB = 16
RPW = INDEX // NSUB


def _fused_kernel(a_ref, x_ref, idx_ref, ws_ref, att_ref):
    bh = pl.program_id(0)
    r = pl.program_id(1)
    h = lax.rem(bh, HEAD_NUM)
    blk = a_ref[0]
    row0 = r * ROWS_BLK
    i_idx = lax.broadcasted_iota(jnp.int32, (ROWS_BLK, S), 0) + row0
    j_idx = lax.broadcasted_iota(jnp.int32, (ROWS_BLK, S), 1)
    masked = jnp.where(i_idx == j_idx, 0.0, blk)
    contrib = jnp.sum(masked, axis=0, keepdims=True)

    @pl.when(jnp.logical_and(h == 0, r == 0))
    def _():
        att_ref[...] = jnp.zeros_like(att_ref)

    att_ref[...] += contrib

    @pl.when(jnp.logical_and(h == HEAD_NUM - 1, r == S // ROWS_BLK - 1))
    def _select():
        a = att_ref[...] * (1.0 / HEAD_NUM)
        jvec = lax.broadcasted_iota(jnp.int32, (1, S), 1)
        valid = jvec >= 1

        u = lax.bitcast_convert_type(a, jnp.uint32)
        key = jnp.where((u >> 31) == 1, ~u, u | jnp.uint32(0x80000000))
        key = jnp.where(valid, key, jnp.uint32(0))

        def body(k, t):
            cand = t | (jnp.uint32(1) << (jnp.uint32(31) - k.astype(jnp.uint32)))
            cnt = jnp.sum((key >= cand).astype(jnp.int32))
            return jnp.where(cnt >= K_TOP, cand, t)

        thr = lax.fori_loop(0, 32, body, jnp.uint32(0))

        def prefix_excl(v):
            acc = v
            for k in (1, 2, 4, 8, 16, 32, 64, 128, 256, 512, 1024):
                acc = acc + jnp.where(jvec >= k, pltpu.roll(acc, k, 1), 0.0)
            return acc - v

        gt = jnp.logical_and(key > thr, valid)
        eq = jnp.logical_and(key == thr, valid)
        n_gt = jnp.sum(gt.astype(jnp.int32))
        need_eq = K_TOP - n_gt
        eq_f = eq.astype(jnp.float32)
        eq_pref = prefix_excl(eq_f)
        sel_rest = jnp.logical_or(
            gt, jnp.logical_and(eq, eq_pref < need_eq.astype(jnp.float32))
        )
        sel_full = jnp.logical_or(sel_rest, jvec == 0)

        sel_f = sel_full.astype(jnp.float32)
        pos = prefix_excl(sel_f)

        other = jnp.logical_and(valid, jnp.logical_not(sel_rest))
        m = jnp.max(jnp.where(other, a, -jnp.inf))
        e = jnp.where(other, jnp.exp(a - m), 0.0)
        w = e / (jnp.sum(e) * N_OTHER)
        xv = x_ref[0]
        x_hi = xv.astype(jnp.bfloat16)
        x_lo = (xv - x_hi.astype(jnp.float32)).astype(jnp.bfloat16)
        dims = (((1,), (0,)), ((), ()))
        ws_ref[0] = lax.dot_general(
            w, x_hi, dims, preferred_element_type=jnp.float32
        ) + lax.dot_general(w, x_lo, dims, preferred_element_type=jnp.float32)

        prow = lax.broadcasted_iota(jnp.int32, (INDEX, S), 0)
        onehot = jnp.logical_and(prow == pos.astype(jnp.int32), sel_full)
        jhi = (jvec >> 7).astype(jnp.float32)
        jlo = (jvec & 127).astype(jnp.float32)
        jmat = jnp.concatenate([jhi, jlo], axis=0)
        hl = lax.dot_general(
            jmat, onehot.astype(jnp.float32),
            (((1,), (1,)), ((), ())),
            preferred_element_type=jnp.float32,
        )
        idx_ref[0] = (hl[0:1] * 128.0 + hl[1:2]).astype(jnp.int32)


def _select(atten, x):
    return pl.pallas_call(
        _fused_kernel,
        grid=(B * HEAD_NUM, S // ROWS_BLK),
        in_specs=[
            pl.BlockSpec((1, ROWS_BLK, S), lambda bh, r: (bh, r, 0)),
            pl.BlockSpec((1, S, D), lambda bh, r: (bh // HEAD_NUM, 0, 0)),
        ],
        out_specs=(
            pl.BlockSpec((1, 1, INDEX), lambda bh, r: (bh // HEAD_NUM, 0, 0)),
            pl.BlockSpec((1, 1, D), lambda bh, r: (bh // HEAD_NUM, 0, 0)),
        ),
        out_shape=(
            jax.ShapeDtypeStruct((B, 1, INDEX), jnp.int32),
            jax.ShapeDtypeStruct((B, 1, D), jnp.float32),
        ),
        scratch_shapes=[pltpu.VMEM((1, S), jnp.float32)],
        compiler_params=pltpu.CompilerParams(
            dimension_semantics=("arbitrary", "arbitrary"),
        ),
    )(atten, x)


def _sc_gather(idx, ws, x):
    mesh = plsc.VectorSubcoreMesh(core_axis_name="c", subcore_axis_name="s")

    @functools.partial(
        pl.kernel,
        out_type=jax.ShapeDtypeStruct((B, INDEX, D), jnp.float32),
        mesh=mesh,
        scratch_types=[
            pltpu.VMEM((RPW,), jnp.int32),
            pltpu.VMEM((RPW, D), jnp.float32),
            pltpu.SemaphoreType.DMA,
        ],
    )
    def sc_kernel(idx_hbm, ws_hbm, x_hbm, out_hbm, my_idx, rows_v, sem):
        c = lax.axis_index("c")
        s = lax.axis_index("s")
        base = s * RPW
        pltpu.sync_copy(idx_hbm.at[c, 0, pl.ds(base, RPW)], my_idx)
        pltpu.async_copy(x_hbm.at[c].at[my_idx], rows_v, sem).wait()

        @pl.when(s == NSUB - 1)
        def _():
            pltpu.sync_copy(ws_hbm.at[c, 0], rows_v.at[RPW - 1])

        pltpu.sync_copy(rows_v, out_hbm.at[c, pl.ds(base, RPW)])

    return sc_kernel(idx, ws, x)


@jax.jit
def kernel(x, atten):
    idx, ws = _select(atten, x)
    return _sc_gather(idx, ws, x)
